# baseline (speedup 1.0000x reference)
import math

import jax
import jax.numpy as jnp
from jax import lax
from jax.experimental import pallas as pl
from jax.experimental.pallas import tpu as pltpu
from jax.experimental.pallas import tpu_sc as plsc

N = 10000
E = 320000
C = 128
H = 16
L = 16
NC = 2
NS = 16
CPC = C // NC
W = 3 * CPC
XW = CPC + L
B = 16
EPT = E // NS
NIT = EPT // B
RPT = 632
RLAST = N - (NS - 1) * RPT

_C1 = math.sqrt(3.0 / (4.0 * math.pi))


def _sc_body(xcat_hbm, pos_hbm, eij_hbm, w1_hbm, b1_hbm,
             w2a_hbm, w2b_hbm, b2a_hbm, b2b_hbm, z_hbm,
             out_hbm,
             w1_v, b1_v, b2_v, tsort_v, tabA_v, tabB_v,
             eij_v, x_v, pi_v, msg_v,
             sem_id, sem_pi, sem_x, sem_sc, acc):
  core = lax.axis_index("c")
  sid = lax.axis_index("s")

  pltpu.sync_copy(w1_hbm, w1_v)
  pltpu.sync_copy(b1_hbm, b1_v)

  @pl.when(core == 0)
  def _():
    pltpu.sync_copy(w2a_hbm, msg_v.at[:, pl.ds(0, CPC)])
    pltpu.sync_copy(b2a_hbm, b2_v)

  @pl.when(core == 1)
  def _():
    pltpu.sync_copy(w2b_hbm, msg_v.at[:, pl.ds(0, CPC)])
    pltpu.sync_copy(b2b_hbm, b2_v)

  @pl.when(sid < NS - 1)
  def _():
    pltpu.sync_copy(z_hbm, acc.at[pl.ds(sid * RPT, RPT)])

  @pl.when(sid == NS - 1)
  def _():
    pltpu.sync_copy(z_hbm.at[pl.ds(0, RLAST)],
                    acc.at[pl.ds((NS - 1) * RPT, RLAST)])

  w1r0 = w1_v[:]
  b1r0 = b1_v[:]
  tbrk = jnp.where(w1r0 == jnp.float32(0.0), jnp.float32(-1e30),
                   -b1r0 / w1r0)
  tbrk = jnp.clip(tbrk, jnp.float32(-1e30), jnp.float32(1e30))
  tsr = lax.sort(tbrk)
  tsort_v[:] = tsr
  for r in range(H + 1):
    if r == 0:
      mid = tsr[0] - jnp.float32(1.0)
    elif r == H:
      mid = tsr[H - 1] + jnp.float32(1.0)
    else:
      mid = tsr[r - 1] * jnp.float32(0.5) + tsr[r] * jnp.float32(0.5)
    act = (mid * w1r0 + b1r0) > jnp.float32(0.0)
    wa = jnp.where(act, w1r0, jnp.float32(0.0))
    ba = jnp.where(act, b1r0, jnp.float32(0.0))
    for cc in range(CPC // L):
      asl = pl.ds(cc * L, L)
      accA = w1r0 * jnp.float32(0.0)
      accB = b2_v[asl]
      for m in range(H):
        w2m = msg_v[m, asl]
        accA = accA + wa[m] * w2m
        accB = accB + ba[m] * w2m
      tabA_v[r, asl] = accA
      tabB_v[r, asl] = accB

  plsc.subcore_barrier()

  iot = lax.iota(jnp.int32, L)
  i3 = iot * 3
  zero16 = iot * 0
  one16 = zero16 + 1
  two16 = zero16 + 2
  ebase0 = sid * EPT

  pltpu.sync_copy(eij_hbm.at[:, pl.ds(ebase0, B)], eij_v.at[0])
  pltpu.async_copy(eij_hbm.at[:, pl.ds(ebase0 + B, B)], eij_v.at[1], sem_id).wait()
  pltpu.async_copy(eij_hbm.at[:, pl.ds(ebase0 + 2 * B, B)], eij_v.at[2], sem_id)

  def launch_pos(s3, s4):
    pltpu.async_copy(pos_hbm.at[eij_v.at[s4, 0]], pi_v.at[s3], sem_pi)

  def launch_x(s2, s4):
    pltpu.async_copy(xcat_hbm.at[core].at[eij_v.at[s4, 1]], x_v.at[s2], sem_x)

  launch_pos(0, 0)
  launch_x(0, 0)
  launch_pos(1, 1)
  launch_x(1, 1)
  pltpu.sync_copy(z_hbm.at[pl.ds(0, L)], msg_v)
  pltpu.async_copy(msg_v, acc.at[iot], sem_sc, add=True)

  def batch(t, carry):
    g = lax.rem(t, 3)
    g2 = lax.rem(t, 2)
    s4 = lax.rem(t, 4)
    g16 = zero16 + g

    pltpu.make_async_copy(pos_hbm.at[pl.ds(0, B)], pi_v.at[g], sem_pi).wait()
    pltpu.make_async_copy(xcat_hbm.at[0, pl.ds(0, B)], x_v.at[g2], sem_x).wait()

    i16 = eij_v[s4, 0, :]

    pltpu.make_async_copy(eij_hbm.at[:, pl.ds(0, B)], eij_v.at[0],
                          sem_id).wait()
    launch_pos(lax.rem(t + 2, 3), lax.rem(t + 2, 4))

    nbase = ebase0 + (t + 3) * B
    pltpu.async_copy(eij_hbm.at[:, pl.ds(nbase, B)],
                     eij_v.at[lax.rem(t + 3, 4)], sem_id)

    ax = plsc.load_gather(pi_v, [g16, iot, zero16])
    ay = plsc.load_gather(pi_v, [g16, iot, one16])
    az = plsc.load_gather(pi_v, [g16, iot, two16])
    g2_16 = zero16 + g2
    bx = plsc.load_gather(x_v, [g2_16, iot, zero16 + CPC])
    by = plsc.load_gather(x_v, [g2_16, iot, zero16 + (CPC + 1)])
    bz = plsc.load_gather(x_v, [g2_16, iot, zero16 + (CPC + 2)])
    vx = ax - bx
    vy = ay - by
    vz = az - bz
    d2 = vx * vx + vy * vy + vz * vz
    d2c = jnp.maximum(d2, jnp.float32(1e-16))
    bits = plsc.bitcast(d2c, jnp.int32)
    y = plsc.bitcast(jnp.int32(0x5F3759DF) - lax.shift_right_logical(bits, 1),
                     jnp.float32)
    for _ in range(3):
      y = y * (jnp.float32(1.5) - jnp.float32(0.5) * d2c * y * y)
    dist16 = d2 * y
    s = y * jnp.float32(_C1)
    sx16 = vx * s
    sy16 = vy * s
    sz16 = vz * s
    tsr16 = tsort_v[:]
    tsc = [tsr16[m] for m in range(H)]
    cmps = [jnp.where(dist16 > tsc[m], jnp.int32(1), jnp.int32(0))
            for m in range(H)]
    while len(cmps) > 1:
      cmps = [cmps[i] + cmps[i + 1] for i in range(0, len(cmps), 2)]
    r16 = cmps[0]

    pltpu.make_async_copy(z_hbm.at[pl.ds(0, L)], msg_v, sem_sc).wait()

    ncc = CPC // L
    cols = [i3 + (cc * L * 3 + k) for cc in range(ncc) for k in range(3)]
    pend = None
    for lane in range(L):
      d = dist16[lane]
      r = r16[lane]
      lane16 = zero16 + lane
      sx = sx16[lane]
      sy = sy16[lane]
      sz = sz16[lane]
      tA = [tabA_v[r, pl.ds(cc * L, L)] for cc in range(ncc)]
      tB = [tabB_v[r, pl.ds(cc * L, L)] for cc in range(ncc)]
      xr = [x_v[g2, lane, pl.ds(cc * L, L)] for cc in range(ncc)]
      rad = [tA[cc] * d + tB[cc] for cc in range(ncc)]
      yc = [xr[cc] * rad[cc] for cc in range(ncc)]
      prods = []
      for cc in range(ncc):
        prods += [yc[cc] * sx, yc[cc] * sy, yc[cc] * sz]
      if pend is not None:
        pl16, pp = pend
        for idx in range(3 * ncc):
          plsc.store_scatter(msg_v, [pl16, cols[idx]], pp[idx])
      pend = (lane16, prods)
    pl16, pp = pend
    for idx in range(3 * ncc):
      plsc.store_scatter(msg_v, [pl16, cols[idx]], pp[idx])

    launch_x(g2, lax.rem(t + 2, 4))

    pltpu.async_copy(msg_v, acc.at[i16], sem_sc, add=True)
    return carry

  lax.fori_loop(0, NIT, batch, 0)
  for _ in range(2):
    pltpu.make_async_copy(pos_hbm.at[pl.ds(0, B)], pi_v.at[0], sem_pi).wait()
    pltpu.make_async_copy(xcat_hbm.at[0, pl.ds(0, B)], x_v.at[0], sem_x).wait()
  pltpu.make_async_copy(eij_hbm.at[:, pl.ds(0, B)], eij_v.at[0], sem_id).wait()
  pltpu.make_async_copy(z_hbm.at[pl.ds(0, L)], msg_v, sem_sc).wait()
  plsc.subcore_barrier()

  @pl.when(jnp.logical_and(core == 0, sid < NS - 1))
  def _():
    pltpu.sync_copy(acc.at[pl.ds(sid * RPT, RPT)],
                    out_hbm.at[0, pl.ds(sid * RPT, RPT)])

  @pl.when(jnp.logical_and(core == 1, sid < NS - 1))
  def _():
    pltpu.sync_copy(acc.at[pl.ds(sid * RPT, RPT)],
                    out_hbm.at[1, pl.ds(sid * RPT, RPT)])

  @pl.when(jnp.logical_and(core == 0, sid == NS - 1))
  def _():
    pltpu.sync_copy(acc.at[pl.ds((NS - 1) * RPT, RLAST)],
                    out_hbm.at[0, pl.ds((NS - 1) * RPT, RLAST)])

  @pl.when(jnp.logical_and(core == 1, sid == NS - 1))
  def _():
    pltpu.sync_copy(acc.at[pl.ds((NS - 1) * RPT, RLAST)],
                    out_hbm.at[1, pl.ds((NS - 1) * RPT, RLAST)])


@jax.jit
def _run(xcat, pos16, eij, w1, b1, w2a, w2b, b2a, b2b, z):
  mesh = plsc.VectorSubcoreMesh(core_axis_name="c", subcore_axis_name="s")
  f = pl.kernel(
      _sc_body,
      mesh=mesh,
      compiler_params=pltpu.CompilerParams(needs_layout_passes=False,
                                           use_tc_tiling_on_sc=False),
      out_type=jax.ShapeDtypeStruct((NC, N, W), jnp.float32),
      scratch_types=[
          pltpu.VMEM((H,), jnp.float32),
          pltpu.VMEM((H,), jnp.float32),
          pltpu.VMEM((CPC,), jnp.float32),
          pltpu.VMEM((H,), jnp.float32),
          pltpu.VMEM((H + 1, CPC), jnp.float32),
          pltpu.VMEM((H + 1, CPC), jnp.float32),
          pltpu.VMEM((4, 2, B), jnp.int32),
          pltpu.VMEM((2, B, XW), jnp.float32),
          pltpu.VMEM((3, B, L), jnp.float32),
          pltpu.VMEM((L, W), jnp.float32),
          pltpu.SemaphoreType.DMA,
          pltpu.SemaphoreType.DMA,
          pltpu.SemaphoreType.DMA,
          pltpu.SemaphoreType.DMA,
          pltpu.VMEM_SHARED((N, W), jnp.float32),
      ],
  )
  return f(xcat, pos16, eij, w1, b1, w2a, w2b, b2a, b2b, z)


def kernel(x, pos, edge_index, W1, b1, W2, b2):
  padc = jnp.zeros((x.shape[0], XW - CPC - 3), jnp.float32)
  xcat = jnp.stack([jnp.concatenate([x[:, :CPC], pos, padc], axis=1),
                    jnp.concatenate([x[:, CPC:], pos, padc], axis=1)])
  pos16 = jnp.pad(pos, ((0, 0), (0, L - 3)))
  w1 = W1.reshape(H)
  w2a = W2[:, :CPC]
  w2b = W2[:, CPC:]
  b2a = b2[:CPC]
  b2b = b2[CPC:]
  z = jnp.zeros((RPT, W), jnp.float32)
  eij = jnp.pad(edge_index, ((0, 0), (0, 3 * B)))
  res = _run(xcat, pos16, eij, w1, b1, w2a, w2b, b2a, b2b, z)
  return res.reshape(NC, N, CPC, 3).transpose(1, 0, 2, 3).reshape(N, C, 3)


# speedup vs baseline: 1.0877x; 1.0877x over previous
import math

import jax
import jax.numpy as jnp
from jax import lax
from jax.experimental import pallas as pl
from jax.experimental.pallas import tpu as pltpu
from jax.experimental.pallas import tpu_sc as plsc

N = 10000
E = 320000
C = 128
H = 16
L = 16
NC = 2
NS = 16
CPC = C // NC
W = 3 * CPC
B = 16
EPT = E // NS
NIT = EPT // B
RPT = 632
RLAST = N - (NS - 1) * RPT

_C1 = math.sqrt(3.0 / (4.0 * math.pi))


def _sc_body(xcat_hbm, pos_hbm, eij_hbm, w1_hbm, b1_hbm,
             w2a_hbm, w2b_hbm, b2a_hbm, b2b_hbm, z_hbm,
             out_hbm,
             w1_v, b1_v, b2_v, tsort_v, tabA_v, tabB_v,
             eij_v, x_v, p_v, msg_v,
             sem_id, sem_p, sem_x, sem_sc, acc):
  core = lax.axis_index("c")
  sid = lax.axis_index("s")

  pltpu.sync_copy(w1_hbm, w1_v)
  pltpu.sync_copy(b1_hbm, b1_v)

  @pl.when(core == 0)
  def _():
    pltpu.sync_copy(w2a_hbm, msg_v.at[:, pl.ds(0, CPC)])
    pltpu.sync_copy(b2a_hbm, b2_v)

  @pl.when(core == 1)
  def _():
    pltpu.sync_copy(w2b_hbm, msg_v.at[:, pl.ds(0, CPC)])
    pltpu.sync_copy(b2b_hbm, b2_v)

  @pl.when(sid < NS - 1)
  def _():
    pltpu.sync_copy(z_hbm, acc.at[pl.ds(sid * RPT, RPT)])

  @pl.when(sid == NS - 1)
  def _():
    pltpu.sync_copy(z_hbm.at[pl.ds(0, RLAST)],
                    acc.at[pl.ds((NS - 1) * RPT, RLAST)])

  w1r0 = w1_v[:]
  b1r0 = b1_v[:]
  tbrk = jnp.where(w1r0 == jnp.float32(0.0), jnp.float32(-1e30),
                   -b1r0 / w1r0)
  tbrk = jnp.clip(tbrk, jnp.float32(-1e30), jnp.float32(1e30))
  tsr = lax.sort(tbrk)
  tsort_v[:] = tsr
  for r in range(H + 1):
    if r == 0:
      mid = tsr[0] - jnp.float32(1.0)
    elif r == H:
      mid = tsr[H - 1] + jnp.float32(1.0)
    else:
      mid = tsr[r - 1] * jnp.float32(0.5) + tsr[r] * jnp.float32(0.5)
    act = (mid * w1r0 + b1r0) > jnp.float32(0.0)
    wa = jnp.where(act, w1r0, jnp.float32(0.0))
    ba = jnp.where(act, b1r0, jnp.float32(0.0))
    for cc in range(CPC // L):
      asl = pl.ds(cc * L, L)
      accA = w1r0 * jnp.float32(0.0)
      accB = b2_v[asl]
      for m in range(H):
        w2m = msg_v[m, asl]
        accA = accA + wa[m] * w2m
        accB = accB + ba[m] * w2m
      tabA_v[r, asl] = accA
      tabB_v[r, asl] = accB

  plsc.subcore_barrier()

  iot = lax.iota(jnp.int32, L)
  i3 = iot * 3
  zero16 = iot * 0
  one16 = zero16 + 1
  two16 = zero16 + 2
  cbase0 = sid * (EPT // L)

  pltpu.sync_copy(eij_hbm.at[cbase0], eij_v.at[0])
  pltpu.async_copy(eij_hbm.at[cbase0 + 1], eij_v.at[1], sem_id).wait()
  pltpu.async_copy(eij_hbm.at[cbase0 + 2], eij_v.at[2], sem_id)

  def launch_pos(s3, s4):
    pltpu.async_copy(pos_hbm.at[eij_v.at[s4]], p_v.at[s3], sem_p)

  def launch_x(s2, s4):
    pltpu.async_copy(xcat_hbm.at[core].at[eij_v.at[s4, pl.ds(L, L)]],
                     x_v.at[s2], sem_x)

  launch_pos(0, 0)
  launch_x(0, 0)
  launch_pos(1, 1)
  launch_x(1, 1)
  pltpu.sync_copy(z_hbm.at[pl.ds(0, L)], msg_v)
  pltpu.async_copy(msg_v, acc.at[iot], sem_sc, add=True)

  def batch(t, carry):
    g = lax.rem(t, 3)
    g2 = lax.rem(t, 2)
    s4 = lax.rem(t, 4)
    g16 = zero16 + g

    pltpu.make_async_copy(pos_hbm.at[pl.ds(0, 2 * B)], p_v.at[g], sem_p).wait()
    pltpu.make_async_copy(xcat_hbm.at[0, pl.ds(0, B)], x_v.at[g2], sem_x).wait()

    i16 = eij_v[s4, pl.ds(0, L)]

    pltpu.make_async_copy(eij_hbm.at[0], eij_v.at[0], sem_id).wait()
    launch_pos(lax.rem(t + 2, 3), lax.rem(t + 2, 4))

    pltpu.async_copy(eij_hbm.at[cbase0 + t + 3],
                     eij_v.at[lax.rem(t + 3, 4)], sem_id)

    jot = iot + L
    ax = plsc.load_gather(p_v, [g16, iot, zero16])
    ay = plsc.load_gather(p_v, [g16, iot, one16])
    az = plsc.load_gather(p_v, [g16, iot, two16])
    bx = plsc.load_gather(p_v, [g16, jot, zero16])
    by = plsc.load_gather(p_v, [g16, jot, one16])
    bz = plsc.load_gather(p_v, [g16, jot, two16])
    vx = ax - bx
    vy = ay - by
    vz = az - bz
    d2 = vx * vx + vy * vy + vz * vz
    d2c = jnp.maximum(d2, jnp.float32(1e-16))
    bits = plsc.bitcast(d2c, jnp.int32)
    y = plsc.bitcast(jnp.int32(0x5F3759DF) - lax.shift_right_logical(bits, 1),
                     jnp.float32)
    for _ in range(3):
      y = y * (jnp.float32(1.5) - jnp.float32(0.5) * d2c * y * y)
    dist16 = d2 * y
    s = y * jnp.float32(_C1)
    sx16 = vx * s
    sy16 = vy * s
    sz16 = vz * s
    tsr16 = tsort_v[:]
    tsc = [tsr16[m] for m in range(H)]
    cmps = [jnp.where(dist16 > tsc[m], jnp.int32(1), jnp.int32(0))
            for m in range(H)]
    while len(cmps) > 1:
      cmps = [cmps[i] + cmps[i + 1] for i in range(0, len(cmps), 2)]
    r16 = cmps[0]

    pltpu.make_async_copy(z_hbm.at[pl.ds(0, L)], msg_v, sem_sc).wait()

    ncc = CPC // L
    cols = [i3 + (cc * L * 3 + k) for cc in range(ncc) for k in range(3)]
    pend = None
    for lane in range(L):
      d = dist16[lane]
      r = r16[lane]
      lane16 = zero16 + lane
      sx = sx16[lane]
      sy = sy16[lane]
      sz = sz16[lane]
      tA = [tabA_v[r, pl.ds(cc * L, L)] for cc in range(ncc)]
      tB = [tabB_v[r, pl.ds(cc * L, L)] for cc in range(ncc)]
      xr = [x_v[g2, lane, pl.ds(cc * L, L)] for cc in range(ncc)]
      rad = [tA[cc] * d + tB[cc] for cc in range(ncc)]
      yc = [xr[cc] * rad[cc] for cc in range(ncc)]
      prods = []
      for cc in range(ncc):
        prods += [yc[cc] * sx, yc[cc] * sy, yc[cc] * sz]
      if pend is not None:
        pl16, pp = pend
        for idx in range(3 * ncc):
          plsc.store_scatter(msg_v, [pl16, cols[idx]], pp[idx])
      pend = (lane16, prods)
    pl16, pp = pend
    for idx in range(3 * ncc):
      plsc.store_scatter(msg_v, [pl16, cols[idx]], pp[idx])

    launch_x(g2, lax.rem(t + 2, 4))

    pltpu.async_copy(msg_v, acc.at[i16], sem_sc, add=True)
    return carry

  lax.fori_loop(0, NIT, batch, 0)
  for _ in range(2):
    pltpu.make_async_copy(pos_hbm.at[pl.ds(0, 2 * B)], p_v.at[0], sem_p).wait()
    pltpu.make_async_copy(xcat_hbm.at[0, pl.ds(0, B)], x_v.at[0], sem_x).wait()
  pltpu.make_async_copy(eij_hbm.at[0], eij_v.at[0], sem_id).wait()
  pltpu.make_async_copy(z_hbm.at[pl.ds(0, L)], msg_v, sem_sc).wait()
  plsc.subcore_barrier()

  @pl.when(jnp.logical_and(core == 0, sid < NS - 1))
  def _():
    pltpu.sync_copy(acc.at[pl.ds(sid * RPT, RPT)],
                    out_hbm.at[0, pl.ds(sid * RPT, RPT)])

  @pl.when(jnp.logical_and(core == 1, sid < NS - 1))
  def _():
    pltpu.sync_copy(acc.at[pl.ds(sid * RPT, RPT)],
                    out_hbm.at[1, pl.ds(sid * RPT, RPT)])

  @pl.when(jnp.logical_and(core == 0, sid == NS - 1))
  def _():
    pltpu.sync_copy(acc.at[pl.ds((NS - 1) * RPT, RLAST)],
                    out_hbm.at[0, pl.ds((NS - 1) * RPT, RLAST)])

  @pl.when(jnp.logical_and(core == 1, sid == NS - 1))
  def _():
    pltpu.sync_copy(acc.at[pl.ds((NS - 1) * RPT, RLAST)],
                    out_hbm.at[1, pl.ds((NS - 1) * RPT, RLAST)])


@jax.jit
def _run(xcat, pos16, eij, w1, b1, w2a, w2b, b2a, b2b, z):
  mesh = plsc.VectorSubcoreMesh(core_axis_name="c", subcore_axis_name="s")
  f = pl.kernel(
      _sc_body,
      mesh=mesh,
      compiler_params=pltpu.CompilerParams(needs_layout_passes=False,
                                           use_tc_tiling_on_sc=False),
      out_type=jax.ShapeDtypeStruct((NC, N, W), jnp.float32),
      scratch_types=[
          pltpu.VMEM((H,), jnp.float32),
          pltpu.VMEM((H,), jnp.float32),
          pltpu.VMEM((CPC,), jnp.float32),
          pltpu.VMEM((H,), jnp.float32),
          pltpu.VMEM((H + 1, CPC), jnp.float32),
          pltpu.VMEM((H + 1, CPC), jnp.float32),
          pltpu.VMEM((4, 2 * B), jnp.int32),
          pltpu.VMEM((2, B, CPC), jnp.float32),
          pltpu.VMEM((3, 2 * B, L), jnp.float32),
          pltpu.VMEM((L, W), jnp.float32),
          pltpu.SemaphoreType.DMA,
          pltpu.SemaphoreType.DMA,
          pltpu.SemaphoreType.DMA,
          pltpu.SemaphoreType.DMA,
          pltpu.VMEM_SHARED((N, W), jnp.float32),
      ],
  )
  return f(xcat, pos16, eij, w1, b1, w2a, w2b, b2a, b2b, z)


def kernel(x, pos, edge_index, W1, b1, W2, b2):
  xcat = jnp.stack([x[:, :CPC], x[:, CPC:]])
  pos16 = jnp.pad(pos, ((0, 0), (0, L - 3)))
  w1 = W1.reshape(H)
  w2a = W2[:, :CPC]
  w2b = W2[:, CPC:]
  b2a = b2[:CPC]
  b2b = b2[CPC:]
  z = jnp.zeros((RPT, W), jnp.float32)
  eijc = jnp.concatenate([edge_index[0].reshape(E // L, L),
                          edge_index[1].reshape(E // L, L)], axis=1)
  eijc = jnp.pad(eijc, ((0, 4), (0, 0)))
  res = _run(xcat, pos16, eijc, w1, b1, w2a, w2b, b2a, b2b, z)
  return res.reshape(NC, N, CPC, 3).transpose(1, 0, 2, 3).reshape(N, C, 3)


# speedup vs baseline: 1.1485x; 1.0558x over previous
import math

import jax
import jax.numpy as jnp
from jax import lax
from jax.experimental import pallas as pl
from jax.experimental.pallas import tpu as pltpu
from jax.experimental.pallas import tpu_sc as plsc

N = 10000
E = 320000
C = 128
H = 16
L = 16
NC = 2
NS = 16
CPC = C // NC
W = 3 * CPC
B = 16
EPT = E // NS
NIT = EPT // B
RPT = 632
RLAST = N - (NS - 1) * RPT

_C1 = math.sqrt(3.0 / (4.0 * math.pi))


def _sc_body(xcat_hbm, pos_hbm, eij_hbm, w1_hbm, b1_hbm,
             w2a_hbm, w2b_hbm, b2a_hbm, b2b_hbm, z_hbm,
             out_hbm,
             w1_v, b1_v, b2_v, tsort_v, tabA_v, tabB_v,
             eij_v, x_v, pi_v, pj_v, msg_v,
             sem_id, sem_pi, sem_pj, sem_x, sem_sc, acc):
  core = lax.axis_index("c")
  sid = lax.axis_index("s")

  pltpu.sync_copy(w1_hbm, w1_v)
  pltpu.sync_copy(b1_hbm, b1_v)

  @pl.when(core == 0)
  def _():
    pltpu.sync_copy(w2a_hbm, msg_v.at[:, pl.ds(0, CPC)])
    pltpu.sync_copy(b2a_hbm, b2_v)

  @pl.when(core == 1)
  def _():
    pltpu.sync_copy(w2b_hbm, msg_v.at[:, pl.ds(0, CPC)])
    pltpu.sync_copy(b2b_hbm, b2_v)

  @pl.when(sid < NS - 1)
  def _():
    pltpu.sync_copy(z_hbm, acc.at[pl.ds(sid * RPT, RPT)])

  @pl.when(sid == NS - 1)
  def _():
    pltpu.sync_copy(z_hbm.at[pl.ds(0, RLAST)],
                    acc.at[pl.ds((NS - 1) * RPT, RLAST)])

  w1r0 = w1_v[:]
  b1r0 = b1_v[:]
  tbrk = jnp.where(w1r0 == jnp.float32(0.0), jnp.float32(-1e30),
                   -b1r0 / w1r0)
  tbrk = jnp.clip(tbrk, jnp.float32(-1e30), jnp.float32(1e30))
  tsr = lax.sort(tbrk)
  tsort_v[:] = tsr
  for r in range(H + 1):
    if r == 0:
      mid = tsr[0] - jnp.float32(1.0)
    elif r == H:
      mid = tsr[H - 1] + jnp.float32(1.0)
    else:
      mid = tsr[r - 1] * jnp.float32(0.5) + tsr[r] * jnp.float32(0.5)
    act = (mid * w1r0 + b1r0) > jnp.float32(0.0)
    wa = jnp.where(act, w1r0, jnp.float32(0.0))
    ba = jnp.where(act, b1r0, jnp.float32(0.0))
    for cc in range(CPC // L):
      asl = pl.ds(cc * L, L)
      accA = w1r0 * jnp.float32(0.0)
      accB = b2_v[asl]
      for m in range(H):
        w2m = msg_v[m, asl]
        accA = accA + wa[m] * w2m
        accB = accB + ba[m] * w2m
      tabA_v[r, asl] = accA
      tabB_v[r, asl] = accB

  plsc.subcore_barrier()

  iot = lax.iota(jnp.int32, L)
  i3 = iot * 3
  zero16 = iot * 0
  one16 = zero16 + 1
  two16 = zero16 + 2
  ebase0 = sid * EPT

  pltpu.sync_copy(eij_hbm.at[:, pl.ds(ebase0, B)], eij_v.at[0])
  pltpu.async_copy(eij_hbm.at[:, pl.ds(ebase0 + B, B)], eij_v.at[1], sem_id).wait()
  pltpu.async_copy(eij_hbm.at[:, pl.ds(ebase0 + 2 * B, B)], eij_v.at[2], sem_id)

  def launch_pos(s3, s4):
    pltpu.async_copy(pos_hbm.at[eij_v.at[s4, 0]], pi_v.at[s3], sem_pi)
    pltpu.async_copy(pos_hbm.at[eij_v.at[s4, 1]], pj_v.at[s3], sem_pj)

  def launch_x(s2, s4):
    pltpu.async_copy(xcat_hbm.at[core].at[eij_v.at[s4, 1]], x_v.at[s2], sem_x)

  launch_pos(0, 0)
  launch_x(0, 0)
  launch_pos(1, 1)
  launch_x(1, 1)
  pltpu.sync_copy(z_hbm.at[pl.ds(0, L)], msg_v)
  pltpu.async_copy(msg_v, acc.at[iot], sem_sc, add=True)

  def batch(t, carry):
    g = lax.rem(t, 3)
    g2 = lax.rem(t, 2)
    s4 = lax.rem(t, 4)
    g16 = zero16 + g

    pltpu.make_async_copy(pos_hbm.at[pl.ds(0, B)], pi_v.at[g], sem_pi).wait()
    pltpu.make_async_copy(pos_hbm.at[pl.ds(0, B)], pj_v.at[g], sem_pj).wait()
    pltpu.make_async_copy(xcat_hbm.at[0, pl.ds(0, B)], x_v.at[g2], sem_x).wait()

    i16 = eij_v[s4, 0, :]

    pltpu.make_async_copy(eij_hbm.at[:, pl.ds(0, B)], eij_v.at[0],
                          sem_id).wait()
    launch_pos(lax.rem(t + 2, 3), lax.rem(t + 2, 4))

    nbase = ebase0 + (t + 3) * B
    pltpu.async_copy(eij_hbm.at[:, pl.ds(nbase, B)],
                     eij_v.at[lax.rem(t + 3, 4)], sem_id)

    ax = plsc.load_gather(pi_v, [g16, iot, zero16])
    ay = plsc.load_gather(pi_v, [g16, iot, one16])
    az = plsc.load_gather(pi_v, [g16, iot, two16])
    bx = plsc.load_gather(pj_v, [g16, iot, zero16])
    by = plsc.load_gather(pj_v, [g16, iot, one16])
    bz = plsc.load_gather(pj_v, [g16, iot, two16])
    vx = ax - bx
    vy = ay - by
    vz = az - bz
    d2 = vx * vx + vy * vy + vz * vz
    d2c = jnp.maximum(d2, jnp.float32(1e-16))
    bits = plsc.bitcast(d2c, jnp.int32)
    y = plsc.bitcast(jnp.int32(0x5F3759DF) - lax.shift_right_logical(bits, 1),
                     jnp.float32)
    for _ in range(3):
      y = y * (jnp.float32(1.5) - jnp.float32(0.5) * d2c * y * y)
    dist16 = d2 * y
    s = y * jnp.float32(_C1)
    sx16 = vx * s
    sy16 = vy * s
    sz16 = vz * s
    tsr16 = tsort_v[:]
    tsc = [tsr16[m] for m in range(H)]
    cmps = [jnp.where(dist16 > tsc[m], jnp.int32(1), jnp.int32(0))
            for m in range(H)]
    while len(cmps) > 1:
      cmps = [cmps[i] + cmps[i + 1] for i in range(0, len(cmps), 2)]
    r16 = cmps[0]

    pltpu.make_async_copy(z_hbm.at[pl.ds(0, L)], msg_v, sem_sc).wait()

    ncc = CPC // L
    cols = [i3 + (cc * L * 3 + k) for cc in range(ncc) for k in range(3)]
    pend = None
    for lane in range(L):
      d = dist16[lane]
      r = r16[lane]
      lane16 = zero16 + lane
      sx = sx16[lane]
      sy = sy16[lane]
      sz = sz16[lane]
      tA = [tabA_v[r, pl.ds(cc * L, L)] for cc in range(ncc)]
      tB = [tabB_v[r, pl.ds(cc * L, L)] for cc in range(ncc)]
      hv0 = x_v[g2, lane, pl.ds(0, 2 * L)]
      hv1 = x_v[g2, lane, pl.ds(2 * L, 2 * L)]
      e0, o0 = plsc.unpack(hv0, format=plsc.PackFormat.INTERLEAVED,
                           preferred_element_type=jnp.float32)
      e1, o1 = plsc.unpack(hv1, format=plsc.PackFormat.INTERLEAVED,
                           preferred_element_type=jnp.float32)
      xr = [e0, o0, e1, o1]
      rad = [tA[cc] * d + tB[cc] for cc in range(ncc)]
      yc = [xr[cc] * rad[cc] for cc in range(ncc)]
      prods = []
      for cc in range(ncc):
        prods += [yc[cc] * sx, yc[cc] * sy, yc[cc] * sz]
      if pend is not None:
        pl16, pp = pend
        for idx in range(3 * ncc):
          plsc.store_scatter(msg_v, [pl16, cols[idx]], pp[idx])
      pend = (lane16, prods)
    pl16, pp = pend
    for idx in range(3 * ncc):
      plsc.store_scatter(msg_v, [pl16, cols[idx]], pp[idx])

    launch_x(g2, lax.rem(t + 2, 4))

    pltpu.async_copy(msg_v, acc.at[i16], sem_sc, add=True)
    return carry

  lax.fori_loop(0, NIT, batch, 0)
  for _ in range(2):
    pltpu.make_async_copy(pos_hbm.at[pl.ds(0, B)], pi_v.at[0], sem_pi).wait()
    pltpu.make_async_copy(pos_hbm.at[pl.ds(0, B)], pj_v.at[0], sem_pj).wait()
    pltpu.make_async_copy(xcat_hbm.at[0, pl.ds(0, B)], x_v.at[0], sem_x).wait()
  pltpu.make_async_copy(eij_hbm.at[:, pl.ds(0, B)], eij_v.at[0], sem_id).wait()
  pltpu.make_async_copy(z_hbm.at[pl.ds(0, L)], msg_v, sem_sc).wait()
  plsc.subcore_barrier()

  @pl.when(jnp.logical_and(core == 0, sid < NS - 1))
  def _():
    pltpu.sync_copy(acc.at[pl.ds(sid * RPT, RPT)],
                    out_hbm.at[0, pl.ds(sid * RPT, RPT)])

  @pl.when(jnp.logical_and(core == 1, sid < NS - 1))
  def _():
    pltpu.sync_copy(acc.at[pl.ds(sid * RPT, RPT)],
                    out_hbm.at[1, pl.ds(sid * RPT, RPT)])

  @pl.when(jnp.logical_and(core == 0, sid == NS - 1))
  def _():
    pltpu.sync_copy(acc.at[pl.ds((NS - 1) * RPT, RLAST)],
                    out_hbm.at[0, pl.ds((NS - 1) * RPT, RLAST)])

  @pl.when(jnp.logical_and(core == 1, sid == NS - 1))
  def _():
    pltpu.sync_copy(acc.at[pl.ds((NS - 1) * RPT, RLAST)],
                    out_hbm.at[1, pl.ds((NS - 1) * RPT, RLAST)])


@jax.jit
def _run(xcat, pos16, eij, w1, b1, w2a, w2b, b2a, b2b, z):
  mesh = plsc.VectorSubcoreMesh(core_axis_name="c", subcore_axis_name="s")
  f = pl.kernel(
      _sc_body,
      mesh=mesh,
      compiler_params=pltpu.CompilerParams(needs_layout_passes=False,
                                           use_tc_tiling_on_sc=False),
      out_type=jax.ShapeDtypeStruct((NC, N, W), jnp.float32),
      scratch_types=[
          pltpu.VMEM((H,), jnp.float32),
          pltpu.VMEM((H,), jnp.float32),
          pltpu.VMEM((CPC,), jnp.float32),
          pltpu.VMEM((H,), jnp.float32),
          pltpu.VMEM((H + 1, CPC), jnp.float32),
          pltpu.VMEM((H + 1, CPC), jnp.float32),
          pltpu.VMEM((4, 2, B), jnp.int32),
          pltpu.VMEM((2, B, CPC), jnp.bfloat16),
          pltpu.VMEM((3, B, L), jnp.float32),
          pltpu.VMEM((3, B, L), jnp.float32),
          pltpu.VMEM((L, W), jnp.float32),
          pltpu.SemaphoreType.DMA,
          pltpu.SemaphoreType.DMA,
          pltpu.SemaphoreType.DMA,
          pltpu.SemaphoreType.DMA,
          pltpu.SemaphoreType.DMA,
          pltpu.VMEM_SHARED((N, W), jnp.float32),
      ],
  )
  return f(xcat, pos16, eij, w1, b1, w2a, w2b, b2a, b2b, z)


def kernel(x, pos, edge_index, W1, b1, W2, b2):
  xcat = jnp.stack([x[:, :CPC], x[:, CPC:]])
  xcat = (xcat.reshape(NC, N, CPC // (2 * L), 2, L)
              .transpose(0, 1, 2, 4, 3)
              .reshape(NC, N, CPC)
              .astype(jnp.bfloat16))
  pos16 = jnp.pad(pos, ((0, 0), (0, L - 3)))
  w1 = W1.reshape(H)
  w2a = W2[:, :CPC]
  w2b = W2[:, CPC:]
  b2a = b2[:CPC]
  b2b = b2[CPC:]
  z = jnp.zeros((RPT, W), jnp.float32)
  eij = jnp.pad(edge_index, ((0, 0), (0, 3 * B)))
  res = _run(xcat, pos16, eij, w1, b1, w2a, w2b, b2a, b2b, z)
  return res.reshape(NC, N, CPC, 3).transpose(1, 0, 2, 3).reshape(N, C, 3)


# speedup vs baseline: 1.1534x; 1.0043x over previous
import math

import jax
import jax.numpy as jnp
from jax import lax
from jax.experimental import pallas as pl
from jax.experimental.pallas import tpu as pltpu
from jax.experimental.pallas import tpu_sc as plsc

N = 10000
E = 320000
C = 128
H = 16
L = 16
NC = 2
NS = 16
CPC = C // NC
W = 3 * CPC
B = 16
EPT = E // NS
NIT = EPT // B
RPT = 632
RLAST = N - (NS - 1) * RPT

_C1 = math.sqrt(3.0 / (4.0 * math.pi))


def _sc_body(xcat_hbm, pos_hbm, eij_hbm, w1_hbm, b1_hbm,
             w2a_hbm, w2b_hbm, b2a_hbm, b2b_hbm, z_hbm,
             out_hbm,
             w1_v, b1_v, b2_v, tsort_v, tabA_v, tabB_v,
             eij_v, x_v, pi_v, pj_v, msg_v,
             sem_id, sem_pi, sem_pj, sem_x, sem_sc, acc):
  core = lax.axis_index("c")
  sid = lax.axis_index("s")

  pltpu.sync_copy(w1_hbm, w1_v)
  pltpu.sync_copy(b1_hbm, b1_v)

  @pl.when(core == 0)
  def _():
    pltpu.sync_copy(w2a_hbm, msg_v.at[:, pl.ds(0, CPC)])
    pltpu.sync_copy(b2a_hbm, b2_v)

  @pl.when(core == 1)
  def _():
    pltpu.sync_copy(w2b_hbm, msg_v.at[:, pl.ds(0, CPC)])
    pltpu.sync_copy(b2b_hbm, b2_v)

  @pl.when(sid < NS - 1)
  def _():
    pltpu.sync_copy(z_hbm, acc.at[pl.ds(sid * RPT, RPT)])

  @pl.when(sid == NS - 1)
  def _():
    pltpu.sync_copy(z_hbm.at[pl.ds(0, RLAST)],
                    acc.at[pl.ds((NS - 1) * RPT, RLAST)])

  w1r0 = w1_v[:]
  b1r0 = b1_v[:]
  tbrk = jnp.where(w1r0 == jnp.float32(0.0), jnp.float32(-1e30),
                   -b1r0 / w1r0)
  tbrk = jnp.clip(tbrk, jnp.float32(-1e30), jnp.float32(1e30))
  tsr = lax.sort(tbrk)
  tsort_v[:] = tsr
  for r in range(H + 1):
    if r == 0:
      mid = tsr[0] - jnp.float32(1.0)
    elif r == H:
      mid = tsr[H - 1] + jnp.float32(1.0)
    else:
      mid = tsr[r - 1] * jnp.float32(0.5) + tsr[r] * jnp.float32(0.5)
    act = (mid * w1r0 + b1r0) > jnp.float32(0.0)
    wa = jnp.where(act, w1r0, jnp.float32(0.0))
    ba = jnp.where(act, b1r0, jnp.float32(0.0))
    for cc in range(CPC // L):
      asl = pl.ds(cc * L, L)
      accA = w1r0 * jnp.float32(0.0)
      accB = b2_v[asl]
      for m in range(H):
        w2m = msg_v[m, asl]
        accA = accA + wa[m] * w2m
        accB = accB + ba[m] * w2m
      tabA_v[r, asl] = accA
      tabB_v[r, asl] = accB

  plsc.subcore_barrier()

  iot = lax.iota(jnp.int32, L)
  i3 = iot * 3
  zero16 = iot * 0
  one16 = zero16 + 1
  two16 = zero16 + 2
  ebase0 = sid * EPT

  pltpu.sync_copy(eij_hbm.at[:, pl.ds(ebase0, B)], eij_v.at[0])
  pltpu.async_copy(eij_hbm.at[:, pl.ds(ebase0 + B, B)], eij_v.at[1], sem_id).wait()
  pltpu.async_copy(eij_hbm.at[:, pl.ds(ebase0 + 2 * B, B)], eij_v.at[2], sem_id)

  def launch_pos(s3, s4):
    pltpu.async_copy(pos_hbm.at[eij_v.at[s4, 0]], pi_v.at[s3], sem_pi)
    pltpu.async_copy(pos_hbm.at[eij_v.at[s4, 1]], pj_v.at[s3], sem_pj)

  def launch_x(s2, s4):
    pltpu.async_copy(xcat_hbm.at[core].at[eij_v.at[s4, 1]], x_v.at[s2], sem_x)

  launch_pos(0, 0)
  launch_x(0, 0)
  launch_pos(1, 1)
  launch_x(1, 1)
  pltpu.sync_copy(z_hbm.at[pl.ds(0, L)], msg_v)
  pltpu.async_copy(msg_v, acc.at[iot], sem_sc, add=True)

  def batch(t, carry):
    g = lax.rem(t, 3)
    g2 = lax.rem(t, 2)
    s4 = lax.rem(t, 4)
    g16 = zero16 + g

    pltpu.make_async_copy(pos_hbm.at[pl.ds(0, B)], pi_v.at[g], sem_pi).wait()
    pltpu.make_async_copy(pos_hbm.at[pl.ds(0, B)], pj_v.at[g], sem_pj).wait()
    pltpu.make_async_copy(xcat_hbm.at[0, pl.ds(0, B)], x_v.at[g2], sem_x).wait()

    i16 = eij_v[s4, 0, :]

    pltpu.make_async_copy(eij_hbm.at[:, pl.ds(0, B)], eij_v.at[0],
                          sem_id).wait()
    launch_pos(lax.rem(t + 2, 3), lax.rem(t + 2, 4))

    nbase = ebase0 + (t + 3) * B
    pltpu.async_copy(eij_hbm.at[:, pl.ds(nbase, B)],
                     eij_v.at[lax.rem(t + 3, 4)], sem_id)

    ax = plsc.load_gather(pi_v, [g16, iot, zero16])
    ay = plsc.load_gather(pi_v, [g16, iot, one16])
    az = plsc.load_gather(pi_v, [g16, iot, two16])
    bx = plsc.load_gather(pj_v, [g16, iot, zero16])
    by = plsc.load_gather(pj_v, [g16, iot, one16])
    bz = plsc.load_gather(pj_v, [g16, iot, two16])
    vx = ax - bx
    vy = ay - by
    vz = az - bz
    d2 = vx * vx + vy * vy + vz * vz
    d2c = jnp.maximum(d2, jnp.float32(1e-16))
    bits = plsc.bitcast(d2c, jnp.int32)
    y = plsc.bitcast(jnp.int32(0x5F3759DF) - lax.shift_right_logical(bits, 1),
                     jnp.float32)
    for _ in range(3):
      y = y * (jnp.float32(1.5) - jnp.float32(0.5) * d2c * y * y)
    dist16 = d2 * y
    s = y * jnp.float32(_C1)
    sx16 = vx * s
    sy16 = vy * s
    sz16 = vz * s
    tsr16 = tsort_v[:]
    tsc = [tsr16[m] for m in range(H)]
    cmps = [jnp.where(dist16 > tsc[m], jnp.int32(1), jnp.int32(0))
            for m in range(H)]
    while len(cmps) > 1:
      cmps = [cmps[i] + cmps[i + 1] for i in range(0, len(cmps), 2)]
    r16 = cmps[0]

    pltpu.make_async_copy(z_hbm.at[pl.ds(0, L)], msg_v, sem_sc).wait()

    ncc = CPC // L
    cols = [i3 + (cc * L * 3 + k) for cc in range(ncc) for k in range(3)]
    pend = None
    for lane in range(L):
      d = dist16[lane]
      r = r16[lane]
      lane16 = zero16 + lane
      sx = sx16[lane]
      sy = sy16[lane]
      sz = sz16[lane]
      tA = [tabA_v[r, pl.ds(cc * L, L)] for cc in range(ncc)]
      tB = [tabB_v[r, pl.ds(cc * L, L)] for cc in range(ncc)]
      xr = [x_v[g2, lane, pl.ds(cc * L, L)] for cc in range(ncc)]
      rad = [tA[cc] * d + tB[cc] for cc in range(ncc)]
      yc = [xr[cc] * rad[cc] for cc in range(ncc)]
      prods = []
      for cc in range(ncc):
        prods += [yc[cc] * sx, yc[cc] * sy, yc[cc] * sz]
      if pend is not None:
        pl16, pp = pend
        for idx in range(3 * ncc):
          plsc.store_scatter(msg_v, [pl16, cols[idx]], pp[idx])
      pend = (lane16, prods)
    pl16, pp = pend
    for idx in range(3 * ncc):
      plsc.store_scatter(msg_v, [pl16, cols[idx]], pp[idx])

    launch_x(g2, lax.rem(t + 2, 4))

    pltpu.async_copy(msg_v, acc.at[i16], sem_sc, add=True)
    return carry

  lax.fori_loop(0, NIT, batch, 0)
  for _ in range(2):
    pltpu.make_async_copy(pos_hbm.at[pl.ds(0, B)], pi_v.at[0], sem_pi).wait()
    pltpu.make_async_copy(pos_hbm.at[pl.ds(0, B)], pj_v.at[0], sem_pj).wait()
    pltpu.make_async_copy(xcat_hbm.at[0, pl.ds(0, B)], x_v.at[0], sem_x).wait()
  pltpu.make_async_copy(eij_hbm.at[:, pl.ds(0, B)], eij_v.at[0], sem_id).wait()
  pltpu.make_async_copy(z_hbm.at[pl.ds(0, L)], msg_v, sem_sc).wait()
  plsc.subcore_barrier()

  @pl.when(jnp.logical_and(core == 0, sid < NS - 1))
  def _():
    pltpu.sync_copy(acc.at[pl.ds(sid * RPT, RPT)],
                    out_hbm.at[0, pl.ds(sid * RPT, RPT)])

  @pl.when(jnp.logical_and(core == 1, sid < NS - 1))
  def _():
    pltpu.sync_copy(acc.at[pl.ds(sid * RPT, RPT)],
                    out_hbm.at[1, pl.ds(sid * RPT, RPT)])

  @pl.when(jnp.logical_and(core == 0, sid == NS - 1))
  def _():
    pltpu.sync_copy(acc.at[pl.ds((NS - 1) * RPT, RLAST)],
                    out_hbm.at[0, pl.ds((NS - 1) * RPT, RLAST)])

  @pl.when(jnp.logical_and(core == 1, sid == NS - 1))
  def _():
    pltpu.sync_copy(acc.at[pl.ds((NS - 1) * RPT, RLAST)],
                    out_hbm.at[1, pl.ds((NS - 1) * RPT, RLAST)])


@jax.jit
def _run(xcat, pos16, eij, w1, b1, w2a, w2b, b2a, b2b, z):
  mesh = plsc.VectorSubcoreMesh(core_axis_name="c", subcore_axis_name="s")
  f = pl.kernel(
      _sc_body,
      mesh=mesh,
      compiler_params=pltpu.CompilerParams(needs_layout_passes=False,
                                           use_tc_tiling_on_sc=False),
      out_type=jax.ShapeDtypeStruct((NC, N, W), jnp.float32),
      scratch_types=[
          pltpu.VMEM((H,), jnp.float32),
          pltpu.VMEM((H,), jnp.float32),
          pltpu.VMEM((CPC,), jnp.float32),
          pltpu.VMEM((H,), jnp.float32),
          pltpu.VMEM((H + 1, CPC), jnp.float32),
          pltpu.VMEM((H + 1, CPC), jnp.float32),
          pltpu.VMEM((4, 2, B), jnp.int32),
          pltpu.VMEM((2, B, CPC), jnp.float32),
          pltpu.VMEM((3, B, L), jnp.float32),
          pltpu.VMEM((3, B, L), jnp.float32),
          pltpu.VMEM((L, W), jnp.float32),
          pltpu.SemaphoreType.DMA,
          pltpu.SemaphoreType.DMA,
          pltpu.SemaphoreType.DMA,
          pltpu.SemaphoreType.DMA,
          pltpu.SemaphoreType.DMA,
          pltpu.VMEM_SHARED((N, W), jnp.float32),
      ],
  )
  return f(xcat, pos16, eij, w1, b1, w2a, w2b, b2a, b2b, z)


def kernel(x, pos, edge_index, W1, b1, W2, b2):
  xcat = jnp.stack([x[:, :CPC], x[:, CPC:]])
  pos16 = jnp.pad(pos, ((0, 0), (0, L - 3)))
  w1 = W1.reshape(H)
  w2a = W2[:, :CPC]
  w2b = W2[:, CPC:]
  b2a = b2[:CPC]
  b2b = b2[CPC:]
  z = jnp.zeros((RPT, W), jnp.float32)
  eij = jnp.pad(edge_index, ((0, 0), (0, 3 * B)))
  res = _run(xcat, pos16, eij, w1, b1, w2a, w2b, b2a, b2b, z)
  return res.reshape(NC, N, CPC, 3).transpose(1, 0, 2, 3).reshape(N, C, 3)


# speedup vs baseline: 1.3017x; 1.1286x over previous
import math

import jax
import jax.numpy as jnp
from jax import lax
from jax.experimental import pallas as pl
from jax.experimental.pallas import tpu as pltpu
from jax.experimental.pallas import tpu_sc as plsc

N = 10000
E = 320000
C = 128
H = 16
L = 16
NC = 2
NS = 16
CPC = C // NC
W = 3 * CPC
B = 16
EB = 32
EPT = E // NS
NIT = EPT // EB
RPT = 632
RLAST = N - (NS - 1) * RPT

_C1 = math.sqrt(3.0 / (4.0 * math.pi))


def _sc_body(xcat_hbm, pos_hbm, eij_hbm, w1_hbm, b1_hbm,
             w2a_hbm, w2b_hbm, b2a_hbm, b2b_hbm, z_hbm,
             out_hbm,
             w1_v, b1_v, b2_v, tsort_v, tabA_v, tabB_v,
             eij_v, x_v, pi_v, pj_v, msg_v,
             sem_id, sem_pi, sem_pj, sem_x, sem_sc, acc):
  core = lax.axis_index("c")
  sid = lax.axis_index("s")

  pltpu.sync_copy(w1_hbm, w1_v)
  pltpu.sync_copy(b1_hbm, b1_v)

  @pl.when(core == 0)
  def _():
    pltpu.sync_copy(w2a_hbm, msg_v.at[:, pl.ds(0, CPC)])
    pltpu.sync_copy(b2a_hbm, b2_v)

  @pl.when(core == 1)
  def _():
    pltpu.sync_copy(w2b_hbm, msg_v.at[:, pl.ds(0, CPC)])
    pltpu.sync_copy(b2b_hbm, b2_v)

  @pl.when(sid < NS - 1)
  def _():
    pltpu.sync_copy(z_hbm, acc.at[pl.ds(sid * RPT, RPT)])

  @pl.when(sid == NS - 1)
  def _():
    pltpu.sync_copy(z_hbm.at[pl.ds(0, RLAST)],
                    acc.at[pl.ds((NS - 1) * RPT, RLAST)])

  w1r0 = w1_v[:]
  b1r0 = b1_v[:]
  tbrk = jnp.where(w1r0 == jnp.float32(0.0), jnp.float32(-1e30),
                   -b1r0 / w1r0)
  tbrk = jnp.clip(tbrk, jnp.float32(-1e30), jnp.float32(1e30))
  tsr = lax.sort(tbrk)
  tsort_v[:] = tsr
  for r in range(H + 1):
    if r == 0:
      mid = tsr[0] - jnp.float32(1.0)
    elif r == H:
      mid = tsr[H - 1] + jnp.float32(1.0)
    else:
      mid = tsr[r - 1] * jnp.float32(0.5) + tsr[r] * jnp.float32(0.5)
    act = (mid * w1r0 + b1r0) > jnp.float32(0.0)
    wa = jnp.where(act, w1r0, jnp.float32(0.0))
    ba = jnp.where(act, b1r0, jnp.float32(0.0))
    for cc in range(CPC // L):
      asl = pl.ds(cc * L, L)
      accA = w1r0 * jnp.float32(0.0)
      accB = b2_v[asl]
      for m in range(H):
        w2m = msg_v[m, asl]
        accA = accA + wa[m] * w2m
        accB = accB + ba[m] * w2m
      tabA_v[r, asl] = accA
      tabB_v[r, asl] = accB

  plsc.subcore_barrier()

  iot = lax.iota(jnp.int32, L)
  i3 = iot * 3
  zero16 = iot * 0
  one16 = zero16 + 1
  two16 = zero16 + 2
  ebase0 = sid * EPT

  pltpu.sync_copy(eij_hbm.at[:, pl.ds(ebase0, EB)], eij_v.at[0])
  pltpu.async_copy(eij_hbm.at[:, pl.ds(ebase0 + EB, EB)], eij_v.at[1], sem_id)

  def launch_pos(s2, si):
    pltpu.async_copy(pos_hbm.at[eij_v.at[si, 0]], pi_v.at[s2], sem_pi)
    pltpu.async_copy(pos_hbm.at[eij_v.at[si, 1]], pj_v.at[s2], sem_pj)

  def launch_x(s2, si):
    pltpu.async_copy(xcat_hbm.at[core].at[eij_v.at[si, 1]], x_v.at[s2], sem_x)

  launch_pos(0, 0)
  launch_x(0, 0)
  pltpu.sync_copy(z_hbm.at[pl.ds(0, L)], msg_v)
  pltpu.async_copy(msg_v, acc.at[iot], sem_sc, add=True)

  def batch(t, carry):
    tb = lax.rem(t, 2)
    nb = 1 - tb
    s3 = lax.rem(t, 3)
    tb16 = zero16 + tb

    pltpu.make_async_copy(pos_hbm.at[pl.ds(0, EB)], pi_v.at[tb], sem_pi).wait()
    pltpu.make_async_copy(pos_hbm.at[pl.ds(0, EB)], pj_v.at[tb], sem_pj).wait()
    pltpu.make_async_copy(xcat_hbm.at[0, pl.ds(0, EB)], x_v.at[tb], sem_x).wait()


    pltpu.make_async_copy(eij_hbm.at[:, pl.ds(0, EB)], eij_v.at[0],
                          sem_id).wait()
    launch_pos(nb, lax.rem(t + 1, 3))
    launch_x(nb, lax.rem(t + 1, 3))

    nbase = ebase0 + (t + 2) * EB
    pltpu.async_copy(eij_hbm.at[:, pl.ds(nbase, EB)],
                     eij_v.at[lax.rem(t + 2, 3)], sem_id)

    ncc = CPC // L
    cols = [i3 + (cc * L * 3 + k) for cc in range(ncc) for k in range(3)]

    def half_body(half, carry2):
      hot = iot + half * L
      i16 = eij_v[s3, 0, pl.ds(half * L, L)]
      ax = plsc.load_gather(pi_v, [tb16, hot, zero16])
      ay = plsc.load_gather(pi_v, [tb16, hot, one16])
      az = plsc.load_gather(pi_v, [tb16, hot, two16])
      bx = plsc.load_gather(pj_v, [tb16, hot, zero16])
      by = plsc.load_gather(pj_v, [tb16, hot, one16])
      bz = plsc.load_gather(pj_v, [tb16, hot, two16])
      vx = ax - bx
      vy = ay - by
      vz = az - bz
      d2 = vx * vx + vy * vy + vz * vz
      d2c = jnp.maximum(d2, jnp.float32(1e-16))
      bits = plsc.bitcast(d2c, jnp.int32)
      y = plsc.bitcast(jnp.int32(0x5F3759DF) - lax.shift_right_logical(bits, 1),
                       jnp.float32)
      for _ in range(3):
        y = y * (jnp.float32(1.5) - jnp.float32(0.5) * d2c * y * y)
      dist16 = d2 * y
      s = y * jnp.float32(_C1)
      sx16 = vx * s
      sy16 = vy * s
      sz16 = vz * s
      tsr16 = tsort_v[:]
      tsc = [tsr16[m] for m in range(H)]
      cmps = [jnp.where(dist16 > tsc[m], jnp.int32(1), jnp.int32(0))
              for m in range(H)]
      while len(cmps) > 1:
        cmps = [cmps[i] + cmps[i + 1] for i in range(0, len(cmps), 2)]
      r16 = cmps[0]

      pltpu.make_async_copy(z_hbm.at[pl.ds(0, L)], msg_v, sem_sc).wait()

      for lane in range(L):
        d = dist16[lane]
        r = r16[lane]
        lane16 = zero16 + lane
        sx = sx16[lane]
        sy = sy16[lane]
        sz = sz16[lane]
        row = half * L + lane
        tA = [tabA_v[r, pl.ds(cc * L, L)] for cc in range(ncc)]
        tB = [tabB_v[r, pl.ds(cc * L, L)] for cc in range(ncc)]
        hv0 = x_v[tb, row, pl.ds(0, 2 * L)]
        hv1 = x_v[tb, row, pl.ds(2 * L, 2 * L)]
        e0, o0 = plsc.unpack(hv0, format=plsc.PackFormat.INTERLEAVED,
                             preferred_element_type=jnp.float32)
        e1, o1 = plsc.unpack(hv1, format=plsc.PackFormat.INTERLEAVED,
                             preferred_element_type=jnp.float32)
        xr = [e0, o0, e1, o1]
        rad = [tA[cc] * d + tB[cc] for cc in range(ncc)]
        yc = [xr[cc] * rad[cc] for cc in range(ncc)]
        for cc in range(ncc):
          plsc.store_scatter(msg_v, [lane16, cols[3 * cc]], yc[cc] * sx)
          plsc.store_scatter(msg_v, [lane16, cols[3 * cc + 1]], yc[cc] * sy)
          plsc.store_scatter(msg_v, [lane16, cols[3 * cc + 2]], yc[cc] * sz)

      pltpu.async_copy(msg_v, acc.at[i16], sem_sc, add=True)
      return carry2

    lax.fori_loop(0, 2, half_body, 0)
    return carry

  lax.fori_loop(0, NIT, batch, 0)
  pltpu.make_async_copy(pos_hbm.at[pl.ds(0, EB)], pi_v.at[0], sem_pi).wait()
  pltpu.make_async_copy(pos_hbm.at[pl.ds(0, EB)], pj_v.at[0], sem_pj).wait()
  pltpu.make_async_copy(xcat_hbm.at[0, pl.ds(0, EB)], x_v.at[0], sem_x).wait()
  pltpu.make_async_copy(eij_hbm.at[:, pl.ds(0, EB)], eij_v.at[0], sem_id).wait()
  pltpu.make_async_copy(z_hbm.at[pl.ds(0, L)], msg_v, sem_sc).wait()
  plsc.subcore_barrier()

  @pl.when(jnp.logical_and(core == 0, sid < NS - 1))
  def _():
    pltpu.sync_copy(acc.at[pl.ds(sid * RPT, RPT)],
                    out_hbm.at[0, pl.ds(sid * RPT, RPT)])

  @pl.when(jnp.logical_and(core == 1, sid < NS - 1))
  def _():
    pltpu.sync_copy(acc.at[pl.ds(sid * RPT, RPT)],
                    out_hbm.at[1, pl.ds(sid * RPT, RPT)])

  @pl.when(jnp.logical_and(core == 0, sid == NS - 1))
  def _():
    pltpu.sync_copy(acc.at[pl.ds((NS - 1) * RPT, RLAST)],
                    out_hbm.at[0, pl.ds((NS - 1) * RPT, RLAST)])

  @pl.when(jnp.logical_and(core == 1, sid == NS - 1))
  def _():
    pltpu.sync_copy(acc.at[pl.ds((NS - 1) * RPT, RLAST)],
                    out_hbm.at[1, pl.ds((NS - 1) * RPT, RLAST)])


@jax.jit
def _run(xcat, pos16, eij, w1, b1, w2a, w2b, b2a, b2b, z):
  mesh = plsc.VectorSubcoreMesh(core_axis_name="c", subcore_axis_name="s")
  f = pl.kernel(
      _sc_body,
      mesh=mesh,
      compiler_params=pltpu.CompilerParams(needs_layout_passes=False,
                                           use_tc_tiling_on_sc=False),
      out_type=jax.ShapeDtypeStruct((NC, N, W), jnp.float32),
      scratch_types=[
          pltpu.VMEM((H,), jnp.float32),
          pltpu.VMEM((H,), jnp.float32),
          pltpu.VMEM((CPC,), jnp.float32),
          pltpu.VMEM((H,), jnp.float32),
          pltpu.VMEM((H + 1, CPC), jnp.float32),
          pltpu.VMEM((H + 1, CPC), jnp.float32),
          pltpu.VMEM((3, 2, EB), jnp.int32),
          pltpu.VMEM((2, EB, CPC), jnp.bfloat16),
          pltpu.VMEM((2, EB, L), jnp.float32),
          pltpu.VMEM((2, EB, L), jnp.float32),
          pltpu.VMEM((L, W), jnp.float32),
          pltpu.SemaphoreType.DMA,
          pltpu.SemaphoreType.DMA,
          pltpu.SemaphoreType.DMA,
          pltpu.SemaphoreType.DMA,
          pltpu.SemaphoreType.DMA,
          pltpu.VMEM_SHARED((N, W), jnp.float32),
      ],
  )
  return f(xcat, pos16, eij, w1, b1, w2a, w2b, b2a, b2b, z)


def kernel(x, pos, edge_index, W1, b1, W2, b2):
  xcat = jnp.stack([x[:, :CPC], x[:, CPC:]])
  xcat = (xcat.reshape(NC, N, CPC // (2 * L), 2, L)
              .transpose(0, 1, 2, 4, 3)
              .reshape(NC, N, CPC)
              .astype(jnp.bfloat16))
  pos16 = jnp.pad(pos, ((0, 0), (0, L - 3)))
  w1 = W1.reshape(H)
  w2a = W2[:, :CPC]
  w2b = W2[:, CPC:]
  b2a = b2[:CPC]
  b2b = b2[CPC:]
  z = jnp.zeros((RPT, W), jnp.float32)
  eij = jnp.pad(edge_index, ((0, 0), (0, 2 * EB)))
  res = _run(xcat, pos16, eij, w1, b1, w2a, w2b, b2a, b2b, z)
  return res.reshape(NC, N, CPC, 3).transpose(1, 0, 2, 3).reshape(N, C, 3)
